# Initial kernel scaffold; baseline (speedup 1.0000x reference)
import jax
import jax.numpy as jnp
from jax.experimental import pallas as pl


def kernel(x, edge_index, batch, W1, b1, W2, b2, W3, b3, Wl, bl):
    raise NotImplementedError("write your pallas kernel here")


# speedup vs baseline: 10.6967x; 10.6967x over previous
import functools

import jax
import jax.numpy as jnp
from jax import lax
from jax.experimental import pallas as pl
from jax.experimental.pallas import tpu as pltpu
from jax.experimental.pallas import tpu_sc as plsc

N = 10000
E = 320000
DIN = 128
H = 128
DOUT = 10
G = 64

NC = 2
NS = 16
NW = NC * NS
CHUNK = 80
ROWS_A = 624
ROWS_LAST = N - (NS - 1) * ROWS_A
EPT = E // NW
EPT_DEG = E // NS

_mesh = plsc.VectorSubcoreMesh(core_axis_name="c", subcore_axis_name="s")


@functools.partial(
    pl.kernel,
    out_type=jax.ShapeDtypeStruct((N,), jnp.float32),
    mesh=_mesh,
    scratch_types=[
        pltpu.VMEM((CHUNK,), jnp.int32),
        pltpu.VMEM((CHUNK,), jnp.float32),
        pltpu.VMEM_SHARED((N,), jnp.float32),
    ],
)
def _deg_sc(dst_hbm, zeros_hbm, out_hbm, idx_v, ones_v, acc):
    c = lax.axis_index("c")
    s = lax.axis_index("s")
    for j in range(CHUNK // 16):
        ones_v[pl.ds(j * 16, 16)] = jnp.full((16,), 1.0, jnp.float32)

    @pl.when(jnp.logical_and(c == 0, s == 0))
    def _():
        pltpu.sync_copy(zeros_hbm, acc)

    plsc.subcore_barrier()

    @pl.when(c == 0)
    def _():
        base = s * EPT_DEG

        def body(i, carry):
            off = base + i * CHUNK
            pltpu.sync_copy(dst_hbm.at[pl.ds(off, CHUNK)], idx_v)
            pltpu.sync_copy(ones_v, acc.at[idx_v], add=True)
            return carry

        lax.fori_loop(0, EPT_DEG // CHUNK, body, 0)

    plsc.subcore_barrier()

    @pl.when(jnp.logical_and(c == 0, s == 0))
    def _():
        pltpu.sync_copy(acc, out_hbm)


@functools.partial(
    pl.kernel,
    out_type=jax.ShapeDtypeStruct((NC, N, H), jnp.float32),
    mesh=_mesh,
    scratch_types=[
        pltpu.VMEM((CHUNK,), jnp.int32),
        pltpu.VMEM((CHUNK,), jnp.int32),
        pltpu.VMEM((CHUNK, H), jnp.float32),
        pltpu.VMEM_SHARED((N, H), jnp.float32),
        pltpu.SemaphoreType.DMA,
    ],
)
def _agg_sc(hs_hbm, src_hbm, dst_hbm, zeros_hbm, out_hbm,
            sidx, didx, rows, acc, sem):
    c = lax.axis_index("c")
    s = lax.axis_index("s")
    rbase = s * ROWS_A

    @pl.when(s < NS - 1)
    def _():
        pltpu.sync_copy(zeros_hbm.at[pl.ds(0, ROWS_A)],
                        acc.at[pl.ds(rbase, ROWS_A)])

    @pl.when(s == NS - 1)
    def _():
        pltpu.sync_copy(zeros_hbm, acc.at[pl.ds(rbase, ROWS_LAST)])

    plsc.subcore_barrier()

    base = (c * NS + s) * EPT

    def body(i, carry):
        off = base + i * CHUNK
        pltpu.sync_copy(src_hbm.at[pl.ds(off, CHUNK)], sidx)
        pltpu.sync_copy(dst_hbm.at[pl.ds(off, CHUNK)], didx)
        pltpu.async_copy(hs_hbm.at[sidx], rows, sem).wait()
        pltpu.sync_copy(rows, acc.at[didx], add=True)
        return carry

    lax.fori_loop(0, EPT // CHUNK, body, 0)
    plsc.subcore_barrier()

    @pl.when(s < NS - 1)
    def _():
        pltpu.sync_copy(acc.at[pl.ds(rbase, ROWS_A)],
                        out_hbm.at[c, pl.ds(rbase, ROWS_A)])

    @pl.when(s == NS - 1)
    def _():
        pltpu.sync_copy(acc.at[pl.ds(rbase, ROWS_LAST)],
                        out_hbm.at[c, pl.ds(rbase, ROWS_LAST)])


RB = 1000


def _tc1_body(dp_ref, x_ref, w_ref, hs_ref, dis_ref):
    deg = dp_ref[...] + 1.0
    dis = lax.rsqrt(deg)
    t = jnp.dot(x_ref[...], w_ref[...], preferred_element_type=jnp.float32)
    hs_ref[...] = dis * t
    dis_ref[...] = dis


def _tc1(dp, x, W1):
    return pl.pallas_call(
        _tc1_body,
        grid=(N // RB,),
        in_specs=[
            pl.BlockSpec((RB, 1), lambda i: (i, 0)),
            pl.BlockSpec((RB, DIN), lambda i: (i, 0)),
            pl.BlockSpec((DIN, H), lambda i: (0, 0)),
        ],
        out_specs=[
            pl.BlockSpec((RB, H), lambda i: (i, 0)),
            pl.BlockSpec((RB, 1), lambda i: (i, 0)),
        ],
        out_shape=[
            jax.ShapeDtypeStruct((N, H), jnp.float32),
            jax.ShapeDtypeStruct((N, 1), jnp.float32),
        ],
    )(dp, x, W1)


def _tc_mid_body(p_ref, hs_ref, dis_ref, b_ref, w_ref, out_ref):
    agg = p_ref[0] + p_ref[1] + hs_ref[...]
    z = dis_ref[...] * agg + b_ref[...]
    a = jnp.maximum(z, 0.0)
    t = jnp.dot(a, w_ref[...], preferred_element_type=jnp.float32)
    out_ref[...] = dis_ref[...] * t


def _tc_mid(p, hs, dis, b, W):
    return pl.pallas_call(
        _tc_mid_body,
        grid=(N // RB,),
        in_specs=[
            pl.BlockSpec((NC, RB, H), lambda i: (0, i, 0)),
            pl.BlockSpec((RB, H), lambda i: (i, 0)),
            pl.BlockSpec((RB, 1), lambda i: (i, 0)),
            pl.BlockSpec((1, H), lambda i: (0, 0)),
            pl.BlockSpec((H, H), lambda i: (0, 0)),
        ],
        out_specs=pl.BlockSpec((RB, H), lambda i: (i, 0)),
        out_shape=jax.ShapeDtypeStruct((N, H), jnp.float32),
    )(p, hs, dis, b, W)


def _tc3_body(p_ref, hs_ref, dis_ref, b_ref, batchf_ref, wl_ref, bl_ref,
              out_ref, sums_scr, cnt_scr):
    i = pl.program_id(0)

    @pl.when(i == 0)
    def _():
        sums_scr[...] = jnp.zeros_like(sums_scr)
        cnt_scr[...] = jnp.zeros_like(cnt_scr)

    agg = p_ref[0] + p_ref[1] + hs_ref[...]
    z = dis_ref[...] * agg + b_ref[...]
    iota_g = lax.broadcasted_iota(jnp.int32, (1, G), 1).astype(jnp.float32)
    onehot = (batchf_ref[...] == iota_g).astype(jnp.float32)
    sums_scr[...] += lax.dot_general(
        onehot, z, (((0,), (0,)), ((), ())),
        preferred_element_type=jnp.float32)
    cnt_scr[...] += lax.dot_general(
        onehot, jnp.ones_like(z), (((0,), (0,)), ((), ())),
        preferred_element_type=jnp.float32)

    @pl.when(i == pl.num_programs(0) - 1)
    def _():
        pooled = sums_scr[...] / jnp.maximum(cnt_scr[...], 1.0)
        out_ref[...] = jnp.dot(pooled, wl_ref[...],
                               preferred_element_type=jnp.float32) + bl_ref[...]


def _tc3(p, hs, dis, b, batchf, Wl, bl):
    return pl.pallas_call(
        _tc3_body,
        grid=(N // RB,),
        in_specs=[
            pl.BlockSpec((NC, RB, H), lambda i: (0, i, 0)),
            pl.BlockSpec((RB, H), lambda i: (i, 0)),
            pl.BlockSpec((RB, 1), lambda i: (i, 0)),
            pl.BlockSpec((1, H), lambda i: (0, 0)),
            pl.BlockSpec((RB, 1), lambda i: (i, 0)),
            pl.BlockSpec((H, DOUT), lambda i: (0, 0)),
            pl.BlockSpec((1, DOUT), lambda i: (0, 0)),
        ],
        out_specs=pl.BlockSpec((G, DOUT), lambda i: (0, 0)),
        out_shape=jax.ShapeDtypeStruct((G, DOUT), jnp.float32),
        scratch_shapes=[
            pltpu.VMEM((G, H), jnp.float32),
            pltpu.VMEM((G, H), jnp.float32),
        ],
    )(p, hs, dis, b, batchf, Wl, bl)


def kernel(x, edge_index, batch, W1, b1, W2, b2, W3, b3, Wl, bl):
    src = edge_index[0]
    dst = edge_index[1]
    zeros_n = jnp.zeros((N,), jnp.float32)
    zeros_rows = jnp.zeros((ROWS_LAST, H), jnp.float32)

    deg = _deg_sc(dst, zeros_n)
    hs1, dis = _tc1(deg.reshape(N, 1), x, W1)
    p = _agg_sc(hs1, src, dst, zeros_rows)
    hs2 = _tc_mid(p, hs1, dis, b1.reshape(1, H), W2)
    p = _agg_sc(hs2, src, dst, zeros_rows)
    hs3 = _tc_mid(p, hs2, dis, b2.reshape(1, H), W3)
    p = _agg_sc(hs3, src, dst, zeros_rows)
    return _tc3(p, hs3, dis, b3.reshape(1, H),
                batch.astype(jnp.float32).reshape(N, 1),
                Wl, bl.reshape(1, DOUT))
